# Initial kernel scaffold; baseline (speedup 1.0000x reference)
import jax
import jax.numpy as jnp
from jax.experimental import pallas as pl


def kernel(x, Wq, Wk, Wv, Wproj, bproj, Wrout, brout, Wnoise, bnoise, We1, be1, We2, be2, g1, b1, g2, b2):
    raise NotImplementedError("write your pallas kernel here")


# speedup vs baseline: 1.3353x; 1.3353x over previous
import functools

import jax
import jax.numpy as jnp
from jax import lax
from jax.experimental import pallas as pl
from jax.experimental.pallas import tpu as pltpu
from jax.experimental.pallas import tpu_sc as plsc

T, C = 2048, 768
H, HS = 12, 64
E, K = 8, 2
FF = 4 * C
BLK = 256
NB = (T * K) // BLK + E
NPAD = NB * BLK
BQ = 512
BT = 512
_HI = jax.lax.Precision.HIGHEST


def _qkv_body(x_ref, w_ref, o_ref):
    o_ref[...] = lax.dot_general(x_ref[...].astype(jnp.bfloat16),
                                 w_ref[...].astype(jnp.bfloat16),
                                 (((1,), (0,)), ((), ())),
                                 preferred_element_type=jnp.float32)


def _qkv(x2d, wqkv):
    return pl.pallas_call(
        _qkv_body,
        grid=(T // BT,),
        in_specs=[pl.BlockSpec((BT, C), lambda i: (i, 0)),
                  pl.BlockSpec((C, 3 * C), lambda i: (0, 0))],
        out_specs=pl.BlockSpec((BT, 3 * C), lambda i: (i, 0)),
        out_shape=jax.ShapeDtypeStruct((T, 3 * C), jnp.float32),
    )(x2d, wqkv)


KB = 1024


def _attn_one(q, k, v, qi):
    qb = q.astype(jnp.bfloat16)
    kb = k.astype(jnp.bfloat16)
    m_old = jnp.full((BQ, 1), -jnp.inf, jnp.float32)
    l_old = jnp.zeros((BQ, 1), jnp.float32)
    acc = jnp.zeros((BQ, HS), jnp.float32)
    for j in range(T // KB):
        kj = lax.slice_in_dim(kb, j * KB, (j + 1) * KB, axis=0)
        vj = lax.slice_in_dim(v, j * KB, (j + 1) * KB, axis=0)
        s = lax.dot_general(qb, kj, (((1,), (1,)), ((), ())),
                            preferred_element_type=jnp.float32) * (C ** -0.5)
        rowi = qi * BQ + lax.broadcasted_iota(jnp.int32, (BQ, KB), 0)
        col = j * KB + lax.broadcasted_iota(jnp.int32, (BQ, KB), 1)
        s = jnp.where(col <= rowi, s, -jnp.inf)
        m_blk = jnp.max(s, axis=1, keepdims=True)
        m_new = jnp.maximum(m_old, m_blk)
        delta = jnp.where(m_old == m_new, 0.0, m_old - m_new)
        p = jnp.exp(s - m_new)
        sum_p = jnp.sum(p, axis=1, keepdims=True)
        el = jnp.exp(delta) * l_old
        l_new = el + sum_p
        pv = lax.dot_general(p.astype(jnp.bfloat16), vj.astype(jnp.bfloat16),
                             (((1,), (0,)), ((), ())),
                             preferred_element_type=jnp.float32)
        acc = (pv + el * acc) * (1.0 / l_new)
        m_old, l_old = m_new, l_new
    return acc


def _attn_body(q_ref, k_ref, v_ref, o_ref):
    qi = pl.program_id(1)
    q = q_ref[...]
    k = k_ref[...]
    v = v_ref[...]
    o_ref[:, :HS] = _attn_one(q[:, :HS], k[:, :HS], v[:, :HS], qi)
    o_ref[:, HS:] = _attn_one(q[:, HS:], k[:, HS:], v[:, HS:], qi)


def _attention(qkv):
    hp = H // 2
    return pl.pallas_call(
        _attn_body,
        grid=(hp, T // BQ),
        in_specs=[pl.BlockSpec((BQ, 2 * HS), lambda h, qi: (qi, h)),
                  pl.BlockSpec((T, 2 * HS), lambda h, qi: (0, hp + h)),
                  pl.BlockSpec((T, 2 * HS), lambda h, qi: (0, 2 * hp + h))],
        out_specs=pl.BlockSpec((BQ, 2 * HS), lambda h, qi: (qi, h)),
        out_shape=jax.ShapeDtypeStruct((T, C), jnp.float32),
    )(qkv, qkv, qkv)


def _ln(v, g, b, eps=1e-5):
    mu = jnp.mean(v, axis=1, keepdims=True)
    var = jnp.mean((v - mu) ** 2, axis=1, keepdims=True)
    return g * (v - mu) * lax.rsqrt(var + eps) + b


def _router_body(att_ref, x_ref, wp_ref, bp_ref, g1_ref, b1_ref,
                 wr_ref, br_ref, wn_ref, bn_ref, nc_ref,
                 x1_ref, gates_ref, sel_ref):
    sa = lax.dot_general(att_ref[...].astype(jnp.bfloat16),
                         wp_ref[...].astype(jnp.bfloat16),
                         (((1,), (0,)), ((), ())),
                         preferred_element_type=jnp.float32)
    sa = sa + bp_ref[...]
    x1 = x_ref[...] + _ln(sa, g1_ref[...], b1_ref[...])
    x1_ref[...] = x1
    x1b = x1.astype(jnp.bfloat16)
    logits = lax.dot_general(x1b, wr_ref[...].astype(jnp.bfloat16),
                             (((1,), (0,)), ((), ())),
                             preferred_element_type=jnp.float32) + br_ref[...]
    nl = lax.dot_general(x1b, wn_ref[...].astype(jnp.bfloat16),
                         (((1,), (0,)), ((), ())),
                         preferred_element_type=jnp.float32) + bn_ref[...]
    sp = jnp.where(nl > 0, nl + jnp.log1p(jnp.exp(-jnp.abs(nl))),
                   jnp.log1p(jnp.exp(nl)))
    noisy = logits + nc_ref[...] * sp
    iota = lax.broadcasted_iota(jnp.int32, (BT, E), 1)
    m1 = jnp.max(noisy, axis=1, keepdims=True)
    i1 = jnp.min(jnp.where(noisy == m1, iota, E), axis=1, keepdims=True)
    noisy2 = jnp.where(iota == i1, -jnp.inf, noisy)
    m2 = jnp.max(noisy2, axis=1, keepdims=True)
    i2 = jnp.min(jnp.where(noisy2 == m2, iota, E), axis=1, keepdims=True)
    sel = (iota == i1) | (iota == i2)
    expg = jnp.where(sel, jnp.exp(noisy - m1), 0.0)
    gates_ref[...] = expg / jnp.sum(expg, axis=1, keepdims=True)
    sel_ref[...] = sel.astype(jnp.int32)


def _router(att2d, x2d, wproj, bproj, g1, b1, wrout, brout, wnoise, bnoise,
            nconst):
    row = lambda i: (i, 0)
    rep = lambda i: (0, 0)
    return pl.pallas_call(
        _router_body,
        grid=(T // BT,),
        in_specs=[pl.BlockSpec((BT, C), row),
                  pl.BlockSpec((BT, C), row),
                  pl.BlockSpec((C, C), rep),
                  pl.BlockSpec((1, C), rep),
                  pl.BlockSpec((1, C), rep),
                  pl.BlockSpec((1, C), rep),
                  pl.BlockSpec((C, E), rep),
                  pl.BlockSpec((1, E), rep),
                  pl.BlockSpec((C, E), rep),
                  pl.BlockSpec((1, E), rep),
                  pl.BlockSpec((BT, E), row)],
        out_specs=[pl.BlockSpec((BT, C), row),
                   pl.BlockSpec((BT, E), row),
                   pl.BlockSpec((BT, E), row)],
        out_shape=[jax.ShapeDtypeStruct((T, C), jnp.float32),
                   jax.ShapeDtypeStruct((T, E), jnp.float32),
                   jax.ShapeDtypeStruct((T, E), jnp.int32)],
    )(att2d, x2d, wproj, bproj, g1, b1, wrout, brout, wnoise, bnoise, nconst)


def _sc_gather(table, idx, chunk):
    n = idx.shape[0]
    d = table.shape[1]
    per_w = n // 32
    nch = per_w // chunk
    mesh = plsc.VectorSubcoreMesh(core_axis_name="c", subcore_axis_name="s")

    @functools.partial(
        pl.kernel, mesh=mesh,
        out_type=jax.ShapeDtypeStruct((n, d), table.dtype),
        scratch_types=[pltpu.VMEM((chunk,), jnp.int32),
                       pltpu.VMEM((chunk, d), table.dtype),
                       pltpu.SemaphoreType.DMA],
    )
    def k(table_hbm, idx_hbm, out_hbm, idx_v, rows_v, sem):
        wid = lax.axis_index("s") * 2 + lax.axis_index("c")
        base = wid * per_w
        for ci in range(nch):
            off = base + ci * chunk
            pltpu.sync_copy(idx_hbm.at[pl.ds(off, chunk)], idx_v)
            pltpu.async_copy(table_hbm.at[idx_v], rows_v, sem).wait()
            pltpu.sync_copy(rows_v, out_hbm.at[pl.ds(off, chunk)])

    return k(table, idx)


def _gmm_body(be_ref, xg_ref, w1_ref, b1_ref, w2_ref, b2_ref, o_ref):
    del be_ref
    h1 = lax.dot_general(xg_ref[...], w1_ref[0], (((1,), (0,)), ((), ())),
                         preferred_element_type=jnp.float32)
    h1 = h1 + b1_ref[0]
    h1 = h1 * 0.5 * (1.0 + lax.erf(h1 * (2.0 ** -0.5)))
    h1 = h1.astype(jnp.bfloat16)
    h2 = lax.dot_general(h1, w2_ref[0], (((1,), (0,)), ((), ())),
                         preferred_element_type=jnp.float32)
    o_ref[...] = h2 + b2_ref[0]


def _gmm(blk_expert, xg, we1, be1, we2, be2):
    grid_spec = pltpu.PrefetchScalarGridSpec(
        num_scalar_prefetch=1,
        grid=(NB,),
        in_specs=[
            pl.BlockSpec((BLK, C), lambda j, be: (j, 0)),
            pl.BlockSpec((1, C, FF), lambda j, be: (be[j], 0, 0)),
            pl.BlockSpec((1, 1, FF), lambda j, be: (be[j], 0, 0)),
            pl.BlockSpec((1, FF, C), lambda j, be: (be[j], 0, 0)),
            pl.BlockSpec((1, 1, C), lambda j, be: (be[j], 0, 0)),
        ],
        out_specs=pl.BlockSpec((BLK, C), lambda j, be: (j, 0)),
    )
    return pl.pallas_call(
        _gmm_body,
        grid_spec=grid_spec,
        out_shape=jax.ShapeDtypeStruct((NPAD, C), jnp.float32),
    )(blk_expert, xg, we1, be1, we2, be2)


def _combine_body(x1_ref, hp_ref, gp_ref, g2_ref, b2_ref, o_ref):
    hp = hp_ref[...]
    moe = gp_ref[:, 0:1] * hp[:, :C] + gp_ref[:, 1:2] * hp[:, C:]
    o_ref[...] = x1_ref[...] + _ln(moe, g2_ref[...], b2_ref[...])


def _combine(x1, hp, gpair, g2, b2):
    row = lambda i: (i, 0)
    rep = lambda i: (0, 0)
    return pl.pallas_call(
        _combine_body,
        grid=(T // BT,),
        in_specs=[pl.BlockSpec((BT, C), row),
                  pl.BlockSpec((BT, 2 * C), row),
                  pl.BlockSpec((BT, 2), row),
                  pl.BlockSpec((1, C), rep),
                  pl.BlockSpec((1, C), rep)],
        out_specs=pl.BlockSpec((BT, C), row),
        out_shape=jax.ShapeDtypeStruct((T, C), jnp.float32),
    )(x1, hp, gpair, g2, b2)


def kernel(x, Wq, Wk, Wv, Wproj, bproj, Wrout, brout, Wnoise, bnoise,
           We1, be1, We2, be2, g1, b1, g2, b2):
    x2d = x[0]
    wqkv = jnp.concatenate([Wq.transpose(1, 0, 2).reshape(C, C),
                            Wk.transpose(1, 0, 2).reshape(C, C),
                            Wv.transpose(1, 0, 2).reshape(C, C)], axis=1)
    att2d = _attention(_qkv(x2d, wqkv))
    nconst = jax.random.normal(jax.random.key(42), (1, T, E), jnp.float32)[0]
    x1, gates, sel = _router(att2d, x2d, Wproj, bproj[None, :], g1[None, :],
                             b1[None, :], Wrout, brout[None, :], Wnoise,
                             bnoise[None, :], nconst)

    counts = jnp.sum(sel, axis=0)
    rank = jnp.cumsum(sel, axis=0) - sel
    nb = (counts + BLK - 1) // BLK
    ends = jnp.cumsum(nb)
    start_row = (ends - nb) * BLK
    blk_expert = jnp.minimum(
        jnp.sum((jnp.arange(NB)[:, None] >= ends[None, :]).astype(jnp.int32),
                axis=1), E - 1).astype(jnp.int32)
    dest = start_row[None, :] + rank
    selb = sel.astype(bool)
    tok = jnp.broadcast_to(jnp.arange(T, dtype=jnp.int32)[:, None], (T, E))
    dest_flat = jnp.where(selb, dest, NPAD).reshape(-1)
    src_token = jnp.zeros((NPAD + 1,), jnp.int32).at[dest_flat].set(
        tok.reshape(-1))[:NPAD]
    iota_e = jnp.arange(E)[None, :]
    e_lo = jnp.min(jnp.where(selb, iota_e, E), axis=1)
    e_hi = jnp.max(jnp.where(selb, iota_e, -1), axis=1)
    p_lo = jnp.take_along_axis(dest, e_lo[:, None], axis=1)
    p_hi = jnp.take_along_axis(dest, e_hi[:, None], axis=1)
    g_lo = jnp.take_along_axis(gates, e_lo[:, None], axis=1)
    g_hi = jnp.take_along_axis(gates, e_hi[:, None], axis=1)
    pos_pair = jnp.concatenate([p_lo, p_hi], axis=1).reshape(-1)
    gpair = jnp.concatenate([g_lo, g_hi], axis=1)

    x1b = x1.astype(jnp.bfloat16)
    x1p = lax.bitcast_convert_type(x1b.reshape(T, C // 2, 2),
                                   jnp.int32)
    xgp = _sc_gather(x1p, src_token, 96)
    xg = lax.bitcast_convert_type(xgp, jnp.bfloat16).reshape(NPAD, C)

    h2 = _gmm(blk_expert, xg, We1.astype(jnp.bfloat16), be1[:, None, :],
              We2.astype(jnp.bfloat16), be2[:, None, :])

    hp = _sc_gather(h2, pos_pair.astype(jnp.int32), 128)
    x2 = _combine(x1, hp.reshape(T, 2 * C), gpair, g2[None, :], b2[None, :])
    return x2[None]


# speedup vs baseline: 1.5557x; 1.1651x over previous
import functools

import jax
import jax.numpy as jnp
from jax import lax
from jax.experimental import pallas as pl
from jax.experimental.pallas import tpu as pltpu
from jax.experimental.pallas import tpu_sc as plsc

T, C = 2048, 768
H, HS = 12, 64
E, K = 8, 2
FF = 4 * C
BLK = 256
NB = (T * K) // BLK + E
NPAD = NB * BLK
BQ = 512
BT = 512
_HI = jax.lax.Precision.HIGHEST


def _qkv_body(x_ref, w_ref, o_ref):
    o_ref[...] = lax.dot_general(x_ref[...].astype(jnp.bfloat16),
                                 w_ref[...].astype(jnp.bfloat16),
                                 (((1,), (0,)), ((), ())),
                                 preferred_element_type=jnp.float32)


def _qkv(x2d, wqkv):
    return pl.pallas_call(
        _qkv_body,
        grid=(T // BT,),
        in_specs=[pl.BlockSpec((BT, C), lambda i: (i, 0)),
                  pl.BlockSpec((C, 3 * C), lambda i: (0, 0))],
        out_specs=pl.BlockSpec((BT, 3 * C), lambda i: (i, 0)),
        out_shape=jax.ShapeDtypeStruct((T, 3 * C), jnp.float32),
    )(x2d, wqkv)


KB = 1024


def _attn_one(q, k, v, qi):
    qb = q.astype(jnp.bfloat16)
    kb = k.astype(jnp.bfloat16)
    m_old = jnp.full((BQ, 1), -jnp.inf, jnp.float32)
    l_old = jnp.zeros((BQ, 1), jnp.float32)
    acc = jnp.zeros((BQ, HS), jnp.float32)
    for j in range(T // KB):
        kj = lax.slice_in_dim(kb, j * KB, (j + 1) * KB, axis=0)
        vj = lax.slice_in_dim(v, j * KB, (j + 1) * KB, axis=0)
        s = lax.dot_general(qb, kj, (((1,), (1,)), ((), ())),
                            preferred_element_type=jnp.float32) * (C ** -0.5)
        rowi = qi * BQ + lax.broadcasted_iota(jnp.int32, (BQ, KB), 0)
        col = j * KB + lax.broadcasted_iota(jnp.int32, (BQ, KB), 1)
        s = jnp.where(col <= rowi, s, -jnp.inf)
        m_blk = jnp.max(s, axis=1, keepdims=True)
        m_new = jnp.maximum(m_old, m_blk)
        delta = jnp.where(m_old == m_new, 0.0, m_old - m_new)
        p = jnp.exp(s - m_new)
        sum_p = jnp.sum(p, axis=1, keepdims=True)
        el = jnp.exp(delta) * l_old
        l_new = el + sum_p
        pv = lax.dot_general(p.astype(jnp.bfloat16), vj.astype(jnp.bfloat16),
                             (((1,), (0,)), ((), ())),
                             preferred_element_type=jnp.float32)
        acc = (pv + el * acc) * (1.0 / l_new)
        m_old, l_old = m_new, l_new
    return acc


def _attn_body(q_ref, k_ref, v_ref, o_ref):
    qi = pl.program_id(1)
    q = q_ref[...]
    k = k_ref[...]
    v = v_ref[...]
    o_ref[:, :HS] = _attn_one(q[:, :HS], k[:, :HS], v[:, :HS], qi)
    o_ref[:, HS:] = _attn_one(q[:, HS:], k[:, HS:], v[:, HS:], qi)


def _attention(qkv):
    hp = H // 2
    return pl.pallas_call(
        _attn_body,
        grid=(hp, T // BQ),
        in_specs=[pl.BlockSpec((BQ, 2 * HS), lambda h, qi: (qi, h)),
                  pl.BlockSpec((T, 2 * HS), lambda h, qi: (0, hp + h)),
                  pl.BlockSpec((T, 2 * HS), lambda h, qi: (0, 2 * hp + h))],
        out_specs=pl.BlockSpec((BQ, 2 * HS), lambda h, qi: (qi, h)),
        out_shape=jax.ShapeDtypeStruct((T, C), jnp.float32),
    )(qkv, qkv, qkv)


def _ln(v, g, b, eps=1e-5):
    mu = jnp.mean(v, axis=1, keepdims=True)
    var = jnp.mean((v - mu) ** 2, axis=1, keepdims=True)
    return g * (v - mu) * lax.rsqrt(var + eps) + b


def _router_body(att_ref, x_ref, wp_ref, bp_ref, g1_ref, b1_ref,
                 wr_ref, br_ref, wn_ref, bn_ref, nc_ref,
                 x1_ref, gates_ref, sel_ref):
    sa = lax.dot_general(att_ref[...].astype(jnp.bfloat16),
                         wp_ref[...].astype(jnp.bfloat16),
                         (((1,), (0,)), ((), ())),
                         preferred_element_type=jnp.float32)
    sa = sa + bp_ref[...]
    x1 = x_ref[...] + _ln(sa, g1_ref[...], b1_ref[...])
    x1_ref[...] = x1
    x1b = x1.astype(jnp.bfloat16)
    logits = lax.dot_general(x1b, wr_ref[...].astype(jnp.bfloat16),
                             (((1,), (0,)), ((), ())),
                             preferred_element_type=jnp.float32) + br_ref[...]
    nl = lax.dot_general(x1b, wn_ref[...].astype(jnp.bfloat16),
                         (((1,), (0,)), ((), ())),
                         preferred_element_type=jnp.float32) + bn_ref[...]
    sp = jnp.where(nl > 0, nl + jnp.log1p(jnp.exp(-jnp.abs(nl))),
                   jnp.log1p(jnp.exp(nl)))
    noisy = logits + nc_ref[...] * sp
    iota = lax.broadcasted_iota(jnp.int32, (BT, E), 1)
    m1 = jnp.max(noisy, axis=1, keepdims=True)
    i1 = jnp.min(jnp.where(noisy == m1, iota, E), axis=1, keepdims=True)
    noisy2 = jnp.where(iota == i1, -jnp.inf, noisy)
    m2 = jnp.max(noisy2, axis=1, keepdims=True)
    i2 = jnp.min(jnp.where(noisy2 == m2, iota, E), axis=1, keepdims=True)
    sel = (iota == i1) | (iota == i2)
    expg = jnp.where(sel, jnp.exp(noisy - m1), 0.0)
    gates_ref[...] = expg / jnp.sum(expg, axis=1, keepdims=True)
    sel_ref[...] = sel.astype(jnp.int32)


def _router(att2d, x2d, wproj, bproj, g1, b1, wrout, brout, wnoise, bnoise,
            nconst):
    row = lambda i: (i, 0)
    rep = lambda i: (0, 0)
    return pl.pallas_call(
        _router_body,
        grid=(T // BT,),
        in_specs=[pl.BlockSpec((BT, C), row),
                  pl.BlockSpec((BT, C), row),
                  pl.BlockSpec((C, C), rep),
                  pl.BlockSpec((1, C), rep),
                  pl.BlockSpec((1, C), rep),
                  pl.BlockSpec((1, C), rep),
                  pl.BlockSpec((C, E), rep),
                  pl.BlockSpec((1, E), rep),
                  pl.BlockSpec((C, E), rep),
                  pl.BlockSpec((1, E), rep),
                  pl.BlockSpec((BT, E), row)],
        out_specs=[pl.BlockSpec((BT, C), row),
                   pl.BlockSpec((BT, E), row),
                   pl.BlockSpec((BT, E), row)],
        out_shape=[jax.ShapeDtypeStruct((T, C), jnp.float32),
                   jax.ShapeDtypeStruct((T, E), jnp.float32),
                   jax.ShapeDtypeStruct((T, E), jnp.int32)],
    )(att2d, x2d, wproj, bproj, g1, b1, wrout, brout, wnoise, bnoise, nconst)


def _sc_gather(table, idx, chunk):
    n = idx.shape[0]
    d = table.shape[1]
    per_w = n // 32
    nch = per_w // chunk
    mesh = plsc.VectorSubcoreMesh(core_axis_name="c", subcore_axis_name="s")

    @functools.partial(
        pl.kernel, mesh=mesh,
        out_type=jax.ShapeDtypeStruct((n, d), table.dtype),
        scratch_types=[pltpu.VMEM((chunk,), jnp.int32),
                       pltpu.VMEM((chunk, d), table.dtype),
                       pltpu.SemaphoreType.DMA],
    )
    def k(table_hbm, idx_hbm, out_hbm, idx_v, rows_v, sem):
        wid = lax.axis_index("s") * 2 + lax.axis_index("c")
        base = wid * per_w
        for ci in range(nch):
            off = base + ci * chunk
            pltpu.sync_copy(idx_hbm.at[pl.ds(off, chunk)], idx_v)
            pltpu.async_copy(table_hbm.at[idx_v], rows_v, sem).wait()
            pltpu.sync_copy(rows_v, out_hbm.at[pl.ds(off, chunk)])

    return k(table, idx)


def _gmm_body(be_ref, xg_ref, w1_ref, b1_ref, w2_ref, b2_ref, o_ref):
    del be_ref
    h1 = lax.dot_general(xg_ref[...], w1_ref[0], (((1,), (0,)), ((), ())),
                         preferred_element_type=jnp.float32)
    h1 = h1 + b1_ref[0]
    h1 = h1 * 0.5 * (1.0 + lax.erf(h1 * (2.0 ** -0.5)))
    h1 = h1.astype(jnp.bfloat16)
    h2 = lax.dot_general(h1, w2_ref[0], (((1,), (0,)), ((), ())),
                         preferred_element_type=jnp.float32)
    o_ref[...] = h2 + b2_ref[0]


def _gmm(blk_expert, xg, we1, be1, we2, be2):
    grid_spec = pltpu.PrefetchScalarGridSpec(
        num_scalar_prefetch=1,
        grid=(NB,),
        in_specs=[
            pl.BlockSpec((BLK, C), lambda j, be: (j, 0)),
            pl.BlockSpec((1, C, FF), lambda j, be: (be[j], 0, 0)),
            pl.BlockSpec((1, 1, FF), lambda j, be: (be[j], 0, 0)),
            pl.BlockSpec((1, FF, C), lambda j, be: (be[j], 0, 0)),
            pl.BlockSpec((1, 1, C), lambda j, be: (be[j], 0, 0)),
        ],
        out_specs=pl.BlockSpec((BLK, C), lambda j, be: (j, 0)),
    )
    return pl.pallas_call(
        _gmm_body,
        grid_spec=grid_spec,
        out_shape=jax.ShapeDtypeStruct((NPAD, C), jnp.float32),
    )(blk_expert, xg, we1, be1, we2, be2)


def _combine_body(x1_ref, hp_ref, gp_ref, g2_ref, b2_ref, o_ref):
    hp = hp_ref[...]
    moe = gp_ref[:, 0:1] * hp[:, :C] + gp_ref[:, 1:2] * hp[:, C:]
    o_ref[...] = x1_ref[...] + _ln(moe, g2_ref[...], b2_ref[...])


def _combine(x1, hp, gpair, g2, b2):
    row = lambda i: (i, 0)
    rep = lambda i: (0, 0)
    return pl.pallas_call(
        _combine_body,
        grid=(T // BT,),
        in_specs=[pl.BlockSpec((BT, C), row),
                  pl.BlockSpec((BT, 2 * C), row),
                  pl.BlockSpec((BT, 2), row),
                  pl.BlockSpec((1, C), rep),
                  pl.BlockSpec((1, C), rep)],
        out_specs=pl.BlockSpec((BT, C), row),
        out_shape=jax.ShapeDtypeStruct((T, C), jnp.float32),
    )(x1, hp, gpair, g2, b2)


def kernel(x, Wq, Wk, Wv, Wproj, bproj, Wrout, brout, Wnoise, bnoise,
           We1, be1, We2, be2, g1, b1, g2, b2):
    x2d = x[0]
    wqkv = jnp.concatenate([Wq.transpose(1, 0, 2).reshape(C, C),
                            Wk.transpose(1, 0, 2).reshape(C, C),
                            Wv.transpose(1, 0, 2).reshape(C, C)], axis=1)
    att2d = _attention(_qkv(x2d, wqkv))
    nconst = jax.random.normal(jax.random.key(42), (1, T, E), jnp.float32)[0]
    x1, gates, sel = _router(att2d, x2d, Wproj, bproj[None, :], g1[None, :],
                             b1[None, :], Wrout, brout[None, :], Wnoise,
                             bnoise[None, :], nconst)

    counts = jnp.sum(sel, axis=0)
    rank = jnp.cumsum(sel, axis=0) - sel
    nb = (counts + BLK - 1) // BLK
    ends = jnp.cumsum(nb)
    start_row = (ends - nb) * BLK
    blk_expert = jnp.minimum(
        jnp.sum((jnp.arange(NB)[:, None] >= ends[None, :]).astype(jnp.int32),
                axis=1), E - 1).astype(jnp.int32)
    dest = start_row[None, :] + rank
    selb = sel.astype(bool)
    tok = jnp.broadcast_to(jnp.arange(T, dtype=jnp.int32)[:, None], (T, E))
    dest_flat = jnp.where(selb, dest, NPAD).reshape(-1)
    src_token = (jnp.arange(NPAD + 1, dtype=jnp.int32) % T).at[
        dest_flat].set(tok.reshape(-1))[:NPAD]
    iota_e = jnp.arange(E)[None, :]
    e_lo = jnp.min(jnp.where(selb, iota_e, E), axis=1)
    e_hi = jnp.max(jnp.where(selb, iota_e, -1), axis=1)
    p_lo = jnp.take_along_axis(dest, e_lo[:, None], axis=1)
    p_hi = jnp.take_along_axis(dest, e_hi[:, None], axis=1)
    g_lo = jnp.take_along_axis(gates, e_lo[:, None], axis=1)
    g_hi = jnp.take_along_axis(gates, e_hi[:, None], axis=1)
    pos_pair = jnp.concatenate([p_lo, p_hi], axis=1).reshape(-1)
    gpair = jnp.concatenate([g_lo, g_hi], axis=1)

    x1b = x1.astype(jnp.bfloat16)
    x1p = lax.bitcast_convert_type(x1b.reshape(T, C // 2, 2),
                                   jnp.int32)
    xgp = _sc_gather(x1p, src_token, 96)
    xg = lax.bitcast_convert_type(xgp, jnp.bfloat16).reshape(NPAD, C)

    h2 = _gmm(blk_expert, xg, We1.astype(jnp.bfloat16), be1[:, None, :],
              We2.astype(jnp.bfloat16), be2[:, None, :])

    hp = _sc_gather(h2, pos_pair.astype(jnp.int32), 128)
    x2 = _combine(x1, hp.reshape(T, 2 * C), gpair, g2[None, :], b2[None, :])
    return x2[None]


# speedup vs baseline: 1.5634x; 1.0049x over previous
import functools

import jax
import jax.numpy as jnp
from jax import lax
from jax.experimental import pallas as pl
from jax.experimental.pallas import tpu as pltpu
from jax.experimental.pallas import tpu_sc as plsc

T, C = 2048, 768
H, HS = 12, 64
E, K = 8, 2
FF = 4 * C
BLK = 256
NB = (T * K) // BLK + E
NPAD = NB * BLK
BQ = 512
BT = 512
_HI = jax.lax.Precision.HIGHEST


def _qkv_body(x_ref, w_ref, o_ref):
    o_ref[...] = lax.dot_general(x_ref[...].astype(jnp.bfloat16),
                                 w_ref[...].astype(jnp.bfloat16),
                                 (((1,), (0,)), ((), ())),
                                 preferred_element_type=jnp.float32)


def _qkv(x2d, wqkv):
    return pl.pallas_call(
        _qkv_body,
        grid=(T // BT,),
        in_specs=[pl.BlockSpec((BT, C), lambda i: (i, 0)),
                  pl.BlockSpec((C, 3 * C), lambda i: (0, 0))],
        out_specs=pl.BlockSpec((BT, 3 * C), lambda i: (i, 0)),
        out_shape=jax.ShapeDtypeStruct((T, 3 * C), jnp.float32),
    )(x2d, wqkv)


KB = 1024


def _attn_one(q, k, v, qi):
    qb = q.astype(jnp.bfloat16)
    kb = k.astype(jnp.bfloat16)
    m_old = jnp.full((BQ, 1), -jnp.inf, jnp.float32)
    l_old = jnp.zeros((BQ, 1), jnp.float32)
    acc = jnp.zeros((BQ, HS), jnp.float32)
    for j in range(T // KB):
        kj = lax.slice_in_dim(kb, j * KB, (j + 1) * KB, axis=0)
        vj = lax.slice_in_dim(v, j * KB, (j + 1) * KB, axis=0)
        s = lax.dot_general(qb, kj, (((1,), (1,)), ((), ())),
                            preferred_element_type=jnp.float32) * (C ** -0.5)
        rowi = qi * BQ + lax.broadcasted_iota(jnp.int32, (BQ, KB), 0)
        col = j * KB + lax.broadcasted_iota(jnp.int32, (BQ, KB), 1)
        s = jnp.where(col <= rowi, s, -jnp.inf)
        m_blk = jnp.max(s, axis=1, keepdims=True)
        m_new = jnp.maximum(m_old, m_blk)
        delta = jnp.where(m_old == m_new, 0.0, m_old - m_new)
        p = jnp.exp(s - m_new)
        sum_p = jnp.sum(p, axis=1, keepdims=True)
        el = jnp.exp(delta) * l_old
        l_new = el + sum_p
        pv = lax.dot_general(p.astype(jnp.bfloat16), vj.astype(jnp.bfloat16),
                             (((1,), (0,)), ((), ())),
                             preferred_element_type=jnp.float32)
        acc = (pv + el * acc) * (1.0 / l_new)
        m_old, l_old = m_new, l_new
    return acc


def _attn_body(q_ref, k_ref, v_ref, o_ref):
    qi = pl.program_id(1)
    q = q_ref[...]
    k = k_ref[...]
    v = v_ref[...]
    o_ref[:, :HS] = _attn_one(q[:, :HS], k[:, :HS], v[:, :HS], qi)
    o_ref[:, HS:] = _attn_one(q[:, HS:], k[:, HS:], v[:, HS:], qi)


def _attention(qkv):
    hp = H // 2
    return pl.pallas_call(
        _attn_body,
        grid=(hp, T // BQ),
        in_specs=[pl.BlockSpec((BQ, 2 * HS), lambda h, qi: (qi, h)),
                  pl.BlockSpec((T, 2 * HS), lambda h, qi: (0, hp + h)),
                  pl.BlockSpec((T, 2 * HS), lambda h, qi: (0, 2 * hp + h))],
        out_specs=pl.BlockSpec((BQ, 2 * HS), lambda h, qi: (qi, h)),
        out_shape=jax.ShapeDtypeStruct((T, C), jnp.float32),
    )(qkv, qkv, qkv)


def _ln(v, g, b, eps=1e-5):
    mu = jnp.mean(v, axis=1, keepdims=True)
    var = jnp.mean((v - mu) ** 2, axis=1, keepdims=True)
    return g * (v - mu) * lax.rsqrt(var + eps) + b


def _router_body(att_ref, x_ref, wp_ref, bp_ref, g1_ref, b1_ref,
                 wr_ref, br_ref, wn_ref, bn_ref, nc_ref,
                 x1_ref, gates_ref, sel_ref):
    sa = lax.dot_general(att_ref[...].astype(jnp.bfloat16),
                         wp_ref[...].astype(jnp.bfloat16),
                         (((1,), (0,)), ((), ())),
                         preferred_element_type=jnp.float32)
    sa = sa + bp_ref[...]
    x1 = x_ref[...] + _ln(sa, g1_ref[...], b1_ref[...])
    x1_ref[...] = x1
    x1b = x1.astype(jnp.bfloat16)
    logits = lax.dot_general(x1b, wr_ref[...].astype(jnp.bfloat16),
                             (((1,), (0,)), ((), ())),
                             preferred_element_type=jnp.float32) + br_ref[...]
    nl = lax.dot_general(x1b, wn_ref[...].astype(jnp.bfloat16),
                         (((1,), (0,)), ((), ())),
                         preferred_element_type=jnp.float32) + bn_ref[...]
    sp = jnp.where(nl > 0, nl + jnp.log1p(jnp.exp(-jnp.abs(nl))),
                   jnp.log1p(jnp.exp(nl)))
    noisy = logits + nc_ref[...] * sp
    iota = lax.broadcasted_iota(jnp.int32, (BT, E), 1)
    m1 = jnp.max(noisy, axis=1, keepdims=True)
    i1 = jnp.min(jnp.where(noisy == m1, iota, E), axis=1, keepdims=True)
    noisy2 = jnp.where(iota == i1, -jnp.inf, noisy)
    m2 = jnp.max(noisy2, axis=1, keepdims=True)
    i2 = jnp.min(jnp.where(noisy2 == m2, iota, E), axis=1, keepdims=True)
    sel = (iota == i1) | (iota == i2)
    expg = jnp.where(sel, jnp.exp(noisy - m1), 0.0)
    gates_ref[...] = expg / jnp.sum(expg, axis=1, keepdims=True)
    sel_ref[...] = sel.astype(jnp.int32)


def _router(att2d, x2d, wproj, bproj, g1, b1, wrout, brout, wnoise, bnoise,
            nconst):
    row = lambda i: (i, 0)
    rep = lambda i: (0, 0)
    return pl.pallas_call(
        _router_body,
        grid=(T // BT,),
        in_specs=[pl.BlockSpec((BT, C), row),
                  pl.BlockSpec((BT, C), row),
                  pl.BlockSpec((C, C), rep),
                  pl.BlockSpec((1, C), rep),
                  pl.BlockSpec((1, C), rep),
                  pl.BlockSpec((1, C), rep),
                  pl.BlockSpec((C, E), rep),
                  pl.BlockSpec((1, E), rep),
                  pl.BlockSpec((C, E), rep),
                  pl.BlockSpec((1, E), rep),
                  pl.BlockSpec((BT, E), row)],
        out_specs=[pl.BlockSpec((BT, C), row),
                   pl.BlockSpec((BT, E), row),
                   pl.BlockSpec((BT, E), row)],
        out_shape=[jax.ShapeDtypeStruct((T, C), jnp.float32),
                   jax.ShapeDtypeStruct((T, E), jnp.float32),
                   jax.ShapeDtypeStruct((T, E), jnp.int32)],
    )(att2d, x2d, wproj, bproj, g1, b1, wrout, brout, wnoise, bnoise, nconst)


def _sc_gather(table, idx, chunk):
    n = idx.shape[0]
    d = table.shape[1]
    per_w = n // 32
    nch = per_w // chunk
    mesh = plsc.VectorSubcoreMesh(core_axis_name="c", subcore_axis_name="s")

    @functools.partial(
        pl.kernel, mesh=mesh,
        out_type=jax.ShapeDtypeStruct((n, d), table.dtype),
        scratch_types=[pltpu.VMEM((chunk,), jnp.int32),
                       pltpu.VMEM((chunk, d), table.dtype),
                       pltpu.SemaphoreType.DMA],
    )
    def k(table_hbm, idx_hbm, out_hbm, idx_v, rows_v, sem):
        wid = lax.axis_index("s") * 2 + lax.axis_index("c")
        base = wid * per_w
        for ci in range(nch):
            off = base + ci * chunk
            pltpu.sync_copy(idx_hbm.at[pl.ds(off, chunk)], idx_v)
            pltpu.async_copy(table_hbm.at[idx_v], rows_v, sem).wait()
            pltpu.sync_copy(rows_v, out_hbm.at[pl.ds(off, chunk)])

    return k(table, idx)


def _gmm_body(meta_ref, xg_ref, w1_ref, b1_ref, w2_ref, b2_ref, o_ref):
    @pl.when(pl.program_id(0) < meta_ref[NB])
    def _():
        h1 = lax.dot_general(xg_ref[...], w1_ref[0], (((1,), (0,)), ((), ())),
                             preferred_element_type=jnp.float32)
        h1 = h1 + b1_ref[0]
        h1 = h1 * 0.5 * (1.0 + lax.erf(h1 * (2.0 ** -0.5)))
        h1 = h1.astype(jnp.bfloat16)
        h2 = lax.dot_general(h1, w2_ref[0], (((1,), (0,)), ((), ())),
                             preferred_element_type=jnp.float32)
        o_ref[...] = h2 + b2_ref[0]


def _gmm(blk_expert, xg, we1, be1, we2, be2):
    grid_spec = pltpu.PrefetchScalarGridSpec(
        num_scalar_prefetch=1,
        grid=(NB,),
        in_specs=[
            pl.BlockSpec((BLK, C), lambda j, be: (j, 0)),
            pl.BlockSpec((1, C, FF), lambda j, be: (be[j], 0, 0)),
            pl.BlockSpec((1, 1, FF), lambda j, be: (be[j], 0, 0)),
            pl.BlockSpec((1, FF, C), lambda j, be: (be[j], 0, 0)),
            pl.BlockSpec((1, 1, C), lambda j, be: (be[j], 0, 0)),
        ],
        out_specs=pl.BlockSpec((BLK, C), lambda j, be: (j, 0)),
    )
    return pl.pallas_call(
        _gmm_body,
        grid_spec=grid_spec,
        out_shape=jax.ShapeDtypeStruct((NPAD, C), jnp.float32),
    )(blk_expert, xg, we1, be1, we2, be2)


def _combine_body(x1_ref, hp_ref, gp_ref, g2_ref, b2_ref, o_ref):
    hp = hp_ref[...]
    moe = gp_ref[:, 0:1] * hp[:, :C] + gp_ref[:, 1:2] * hp[:, C:]
    o_ref[...] = x1_ref[...] + _ln(moe, g2_ref[...], b2_ref[...])


def _combine(x1, hp, gpair, g2, b2):
    row = lambda i: (i, 0)
    rep = lambda i: (0, 0)
    return pl.pallas_call(
        _combine_body,
        grid=(T // BT,),
        in_specs=[pl.BlockSpec((BT, C), row),
                  pl.BlockSpec((BT, 2 * C), row),
                  pl.BlockSpec((BT, 2), row),
                  pl.BlockSpec((1, C), rep),
                  pl.BlockSpec((1, C), rep)],
        out_specs=pl.BlockSpec((BT, C), row),
        out_shape=jax.ShapeDtypeStruct((T, C), jnp.float32),
    )(x1, hp, gpair, g2, b2)


def kernel(x, Wq, Wk, Wv, Wproj, bproj, Wrout, brout, Wnoise, bnoise,
           We1, be1, We2, be2, g1, b1, g2, b2):
    x2d = x[0]
    wqkv = jnp.concatenate([Wq.transpose(1, 0, 2).reshape(C, C),
                            Wk.transpose(1, 0, 2).reshape(C, C),
                            Wv.transpose(1, 0, 2).reshape(C, C)], axis=1)
    att2d = _attention(_qkv(x2d, wqkv))
    nconst = jax.random.normal(jax.random.key(42), (1, T, E), jnp.float32)[0]
    x1, gates, sel = _router(att2d, x2d, Wproj, bproj[None, :], g1[None, :],
                             b1[None, :], Wrout, brout[None, :], Wnoise,
                             bnoise[None, :], nconst)

    counts = jnp.sum(sel, axis=0)
    rank = jnp.cumsum(sel, axis=0) - sel
    nb = (counts + BLK - 1) // BLK
    ends = jnp.cumsum(nb)
    start_row = (ends - nb) * BLK
    blk_expert = jnp.minimum(
        jnp.sum((jnp.arange(NB)[:, None] >= ends[None, :]).astype(jnp.int32),
                axis=1), E - 1).astype(jnp.int32)
    blk_meta = jnp.concatenate([blk_expert, ends[E - 1:].astype(jnp.int32)])
    dest = start_row[None, :] + rank
    selb = sel.astype(bool)
    tok = jnp.broadcast_to(jnp.arange(T, dtype=jnp.int32)[:, None], (T, E))
    dest_flat = jnp.where(selb, dest, NPAD).reshape(-1)
    src_token = (jnp.arange(NPAD + 1, dtype=jnp.int32) % T).at[
        dest_flat].set(tok.reshape(-1))[:NPAD]
    iota_e = jnp.arange(E)[None, :]
    e_lo = jnp.min(jnp.where(selb, iota_e, E), axis=1)
    e_hi = jnp.max(jnp.where(selb, iota_e, -1), axis=1)
    p_lo = jnp.take_along_axis(dest, e_lo[:, None], axis=1)
    p_hi = jnp.take_along_axis(dest, e_hi[:, None], axis=1)
    g_lo = jnp.take_along_axis(gates, e_lo[:, None], axis=1)
    g_hi = jnp.take_along_axis(gates, e_hi[:, None], axis=1)
    pos_pair = jnp.concatenate([p_lo, p_hi], axis=1).reshape(-1)
    gpair = jnp.concatenate([g_lo, g_hi], axis=1)

    x1b = x1.astype(jnp.bfloat16)
    x1p = lax.bitcast_convert_type(x1b.reshape(T, C // 2, 2),
                                   jnp.int32)
    xgp = _sc_gather(x1p, src_token, 96)
    xg = lax.bitcast_convert_type(xgp, jnp.bfloat16).reshape(NPAD, C)

    h2 = _gmm(blk_meta, xg, We1.astype(jnp.bfloat16), be1[:, None, :],
              We2.astype(jnp.bfloat16), be2[:, None, :])

    hp = _sc_gather(h2, pos_pair.astype(jnp.int32), 128)
    x2 = _combine(x1, hp.reshape(T, 2 * C), gpair, g2[None, :], b2[None, :])
    return x2[None]
